# Initial kernel scaffold; baseline (speedup 1.0000x reference)
import jax
import jax.numpy as jnp
from jax.experimental import pallas as pl


def kernel(edge_index, x_node, x_trace, x_log, W_node, b_node, W_log, b_log, W_tr_src, W_tr_self, b_tr, ln_node_g, ln_node_b, ln_trace_g, ln_trace_b, ln_log_g, ln_log_b, ffn_node_W1, ffn_node_b1, ffn_node_W2, ffn_node_b2, ln2_node_g, ln2_node_b, ffn_trace_W1, ffn_trace_b1, ffn_trace_W2, ffn_trace_b2, ln2_trace_g, ln2_trace_b, ffn_log_W1, ffn_log_b1, ffn_log_W2, ffn_log_b2, ln2_log_g, ln2_log_b):
    raise NotImplementedError("write your pallas kernel here")


# speedup vs baseline: 3.0754x; 3.0754x over previous
import functools

import jax
import jax.numpy as jnp
from jax import lax
from jax.experimental import pallas as pl
from jax.experimental.pallas import tpu as pltpu
from jax.experimental.pallas import tpu_sc as plsc

N = 10000
E = 320000
D = 128
DFF = 512

NC = 2
NS = 16
CHUNK = 80
NPAD = 10240
ROWS_PER_TILE = NPAD // NS

_MESH = plsc.VectorSubcoreMesh(core_axis_name="c", subcore_axis_name="s")


@functools.partial(
    pl.kernel,
    out_type=jax.ShapeDtypeStruct((2 * NPAD, D), jnp.float32),
    mesh=_MESH,
    scratch_types=[
        pltpu.VMEM((CHUNK,), jnp.int32),
        pltpu.VMEM((CHUNK,), jnp.int32),
        pltpu.VMEM((CHUNK, D), jnp.float32),
        pltpu.VMEM_SHARED((NPAD, D), jnp.float32),
        pltpu.SemaphoreType.DMA,
    ],
)
def _sc_segsum(src_cat, dst, xcat, zeros_nd, agg, sidx, didx, rows, acc, sem):
    c = lax.axis_index("c")
    s = lax.axis_index("s")
    pltpu.sync_copy(zeros_nd.at[pl.ds(s * ROWS_PER_TILE, ROWS_PER_TILE)],
                    acc.at[pl.ds(s * ROWS_PER_TILE, ROWS_PER_TILE)])
    plsc.subcore_barrier()

    ept = E // NS
    sbase = c * E + s * ept
    dbase = s * ept

    def body(i, _):
        pltpu.sync_copy(src_cat.at[pl.ds(sbase + i * CHUNK, CHUNK)], sidx)
        pltpu.sync_copy(dst.at[pl.ds(dbase + i * CHUNK, CHUNK)], didx)
        pltpu.async_copy(xcat.at[sidx], rows, sem).wait()
        pltpu.sync_copy(rows, acc.at[didx], add=True)
        return ()

    lax.fori_loop(0, ept // CHUNK, body, ())
    plsc.subcore_barrier()
    pltpu.sync_copy(acc.at[pl.ds(s * ROWS_PER_TILE, ROWS_PER_TILE)],
                    agg.at[pl.ds(c * NPAD + s * ROWS_PER_TILE, ROWS_PER_TILE)])


@functools.partial(
    pl.kernel,
    out_type=(jax.ShapeDtypeStruct((E, D), jnp.float32),
              jax.ShapeDtypeStruct((E, D), jnp.float32)),
    mesh=_MESH,
    scratch_types=[
        pltpu.VMEM((CHUNK,), jnp.int32),
        pltpu.VMEM((CHUNK,), jnp.int32),
        pltpu.VMEM((CHUNK, D), jnp.float32),
        pltpu.VMEM((CHUNK, D), jnp.float32),
        pltpu.SemaphoreType.DMA,
        pltpu.SemaphoreType.DMA,
    ],
)
def _sc_gather2(src, dst, p, ga, gb, sidx, didx, bufa, bufb, sema, semb):
    c = lax.axis_index("c")
    s = lax.axis_index("s")
    wid = s * NC + c
    epw = E // (NC * NS)
    base = wid * epw

    def body(i, _):
        off = base + i * CHUNK
        pltpu.sync_copy(src.at[pl.ds(off, CHUNK)], sidx)
        pltpu.sync_copy(dst.at[pl.ds(off, CHUNK)], didx)
        cpa = pltpu.async_copy(p.at[sidx], bufa, sema)
        cpb = pltpu.async_copy(p.at[didx], bufb, semb)
        cpa.wait()
        cpb.wait()
        pltpu.sync_copy(bufa, ga.at[pl.ds(off, CHUNK)])
        pltpu.sync_copy(bufb, gb.at[pl.ds(off, CHUNK)])
        return ()

    lax.fori_loop(0, epw // CHUNK, body, ())


def _ln(x, g, b):
    m = jnp.mean(x, axis=-1, keepdims=True)
    xc = x - m
    v = jnp.mean(xc * xc, axis=-1, keepdims=True)
    return xc * lax.rsqrt(v + 1e-5) * g + b


def _ffn_ln(x, w1, b1, w2, b2, g, b):
    y = jnp.dot(x, w1, preferred_element_type=jnp.float32) + b1
    y = jnp.where(y >= 0, y, 0.01 * y)
    y = jnp.dot(y, w2, preferred_element_type=jnp.float32) + b2
    return _ln(x + y, g, b)


def _tc_p_body(x_ref, w_ref, o_ref):
    o_ref[...] = jnp.dot(x_ref[...], w_ref[...],
                         preferred_element_type=jnp.float32)


def _tc_nodelog_body(aggn, xn, aggl, xl,
                     w_node, b_node, w_log, b_log,
                     lnn_g, lnn_b, lnl_g, lnl_b,
                     w1n, b1n, w2n, b2n, g2n, b2n_,
                     w1l, b1l, w2l, b2l, g2l, b2l_,
                     on_ref, ol_ref):
    hn = jnp.dot(aggn[...], w_node[...],
                 preferred_element_type=jnp.float32) + b_node[...]
    xnn = _ln(xn[...] + hn, lnn_g[...], lnn_b[...])
    on_ref[...] = _ffn_ln(xnn, w1n[...], b1n[...], w2n[...], b2n[...],
                          g2n[...], b2n_[...])
    hl = jnp.dot(aggl[...], w_log[...],
                 preferred_element_type=jnp.float32) + b_log[...]
    xll = _ln(xl[...] + hl, lnl_g[...], lnl_b[...])
    ol_ref[...] = _ffn_ln(xll, w1l[...], b1l[...], w2l[...], b2l[...],
                          g2l[...], b2l_[...])


def _tc_trace_body(xt, ga, gb,
                   w_self, b_tr, lnt_g, lnt_b,
                   w1, b1, w2, b2, g2, b2_,
                   o_ref):
    h = ga[...] + gb[...] + jnp.dot(
        xt[...], w_self[...], preferred_element_type=jnp.float32) + b_tr[...]
    x = _ln(xt[...] + h, lnt_g[...], lnt_b[...])
    o_ref[...] = _ffn_ln(x, w1[...], b1[...], w2[...], b2[...],
                         g2[...], b2_[...])


def _row_spec(r, cols):
    return pl.BlockSpec((r, cols), lambda i: (i, 0))


def _full_spec(shape):
    nd = len(shape)
    return pl.BlockSpec(shape, lambda i: (0,) * nd)


def kernel(edge_index, x_node, x_trace, x_log, W_node, b_node, W_log, b_log,
           W_tr_src, W_tr_self, b_tr,
           ln_node_g, ln_node_b, ln_trace_g, ln_trace_b, ln_log_g, ln_log_b,
           ffn_node_W1, ffn_node_b1, ffn_node_W2, ffn_node_b2,
           ln2_node_g, ln2_node_b,
           ffn_trace_W1, ffn_trace_b1, ffn_trace_W2, ffn_trace_b2,
           ln2_trace_g, ln2_trace_b,
           ffn_log_W1, ffn_log_b1, ffn_log_W2, ffn_log_b2,
           ln2_log_g, ln2_log_b):
    src = edge_index[0]
    dst = edge_index[1]
    src_cat = jnp.concatenate([src, src + N])
    xcat = jnp.concatenate([x_node, x_log], axis=0)
    zeros_nd = jnp.zeros((NPAD, D), jnp.float32)

    aggcat = _sc_segsum(src_cat, dst, xcat, zeros_nd)
    agg_node = aggcat[:N]
    agg_log = aggcat[NPAD:NPAD + N]

    RP = 1000
    p = pl.pallas_call(
        _tc_p_body,
        grid=(N // RP,),
        in_specs=[_row_spec(RP, D), _full_spec((D, D))],
        out_specs=_row_spec(RP, D),
        out_shape=jax.ShapeDtypeStruct((N, D), jnp.float32),
        compiler_params=pltpu.CompilerParams(
            dimension_semantics=("parallel",)),
    )(x_node, W_tr_src)

    ga, gb = _sc_gather2(src, dst, p)

    vecs = dict(
        b_node=b_node.reshape(1, D), b_log=b_log.reshape(1, D),
        ln_node_g=ln_node_g.reshape(1, D), ln_node_b=ln_node_b.reshape(1, D),
        ln_log_g=ln_log_g.reshape(1, D), ln_log_b=ln_log_b.reshape(1, D),
        ffn_node_b1=ffn_node_b1.reshape(1, DFF),
        ffn_node_b2=ffn_node_b2.reshape(1, D),
        ln2_node_g=ln2_node_g.reshape(1, D), ln2_node_b=ln2_node_b.reshape(1, D),
        ffn_log_b1=ffn_log_b1.reshape(1, DFF),
        ffn_log_b2=ffn_log_b2.reshape(1, D),
        ln2_log_g=ln2_log_g.reshape(1, D), ln2_log_b=ln2_log_b.reshape(1, D),
    )
    out_node, out_log = pl.pallas_call(
        _tc_nodelog_body,
        grid=(N // RP,),
        in_specs=[
            _row_spec(RP, D), _row_spec(RP, D),
            _row_spec(RP, D), _row_spec(RP, D),
            _full_spec((D, D)), _full_spec((1, D)),
            _full_spec((D, D)), _full_spec((1, D)),
            _full_spec((1, D)), _full_spec((1, D)),
            _full_spec((1, D)), _full_spec((1, D)),
            _full_spec((D, DFF)), _full_spec((1, DFF)),
            _full_spec((DFF, D)), _full_spec((1, D)),
            _full_spec((1, D)), _full_spec((1, D)),
            _full_spec((D, DFF)), _full_spec((1, DFF)),
            _full_spec((DFF, D)), _full_spec((1, D)),
            _full_spec((1, D)), _full_spec((1, D)),
        ],
        out_specs=[_row_spec(RP, D), _row_spec(RP, D)],
        out_shape=[jax.ShapeDtypeStruct((N, D), jnp.float32),
                   jax.ShapeDtypeStruct((N, D), jnp.float32)],
        compiler_params=pltpu.CompilerParams(
            dimension_semantics=("parallel",)),
    )(agg_node, x_node, agg_log, x_log,
      W_node, vecs["b_node"], W_log, vecs["b_log"],
      vecs["ln_node_g"], vecs["ln_node_b"], vecs["ln_log_g"], vecs["ln_log_b"],
      ffn_node_W1, vecs["ffn_node_b1"], ffn_node_W2, vecs["ffn_node_b2"],
      vecs["ln2_node_g"], vecs["ln2_node_b"],
      ffn_log_W1, vecs["ffn_log_b1"], ffn_log_W2, vecs["ffn_log_b2"],
      vecs["ln2_log_g"], vecs["ln2_log_b"])

    RT = 1000
    out_trace = pl.pallas_call(
        _tc_trace_body,
        grid=(E // RT,),
        in_specs=[
            _row_spec(RT, D), _row_spec(RT, D), _row_spec(RT, D),
            _full_spec((D, D)), _full_spec((1, D)),
            _full_spec((1, D)), _full_spec((1, D)),
            _full_spec((D, DFF)), _full_spec((1, DFF)),
            _full_spec((DFF, D)), _full_spec((1, D)),
            _full_spec((1, D)), _full_spec((1, D)),
        ],
        out_specs=_row_spec(RT, D),
        out_shape=jax.ShapeDtypeStruct((E, D), jnp.float32),
        compiler_params=pltpu.CompilerParams(
            dimension_semantics=("parallel",)),
    )(x_trace, ga, gb,
      W_tr_self, b_tr.reshape(1, D),
      ln_trace_g.reshape(1, D), ln_trace_b.reshape(1, D),
      ffn_trace_W1, ffn_trace_b1.reshape(1, DFF),
      ffn_trace_W2, ffn_trace_b2.reshape(1, D),
      ln2_trace_g.reshape(1, D), ln2_trace_b.reshape(1, D))

    return (out_node, out_trace, out_log)
